# baseline (speedup 1.0000x reference)
import functools

import jax
import jax.numpy as jnp
from jax import lax
from jax.experimental import pallas as pl
from jax.experimental.pallas import tpu as pltpu
from jax.experimental.pallas import tpu_sc as plsc

_NW = 32
_L = 16
_W = 8


def _lane_gather(x, idx):
    return lax.gather(
        x, idx[:, None],
        lax.GatherDimensionNumbers(
            offset_dims=(), collapsed_slice_dims=(0,), start_index_map=(0,)),
        slice_sizes=(1,),
        mode=lax.GatherScatterMode.PROMISE_IN_BOUNDS)


def _sc_embed(idx_hbm, pat_hbm, out_hbm, idx_v, dlist_v, pat_v,
              psem, ssem0, ssem1):
    wid = lax.axis_index("s") * 2 + lax.axis_index("c")
    n_rows = out_hbm.shape[0]
    b_per_w = n_rows // _NW
    base = wid * b_per_w
    n_ch = b_per_w // _L

    pcopy = pltpu.async_copy(pat_hbm, pat_v, psem)
    pltpu.sync_copy(idx_hbm.at[pl.ds(base, b_per_w)], idx_v)
    lanes = lax.iota(jnp.int32, _L)
    pcopy.wait()

    scats = [None] * n_ch
    for i in range(n_ch):
        if i >= _W:
            scats[i - _W].wait()
        v = idx_v[pl.ds(i * _L, _L)]
        rows = base + i * _L + lanes
        m0 = 1 - v
        s = m0
        for d in (1, 2, 4, 8):
            sh = _lane_gather(s, jnp.maximum(lanes - d, 0))
            s = s + jnp.where(lanes >= d, sh, 0)
        excl = s - m0
        z = s[_L - 1]
        tgt = jnp.where(v == 0, excl, z + lanes - excl)
        srows = jnp.zeros((_L,), jnp.int32)
        for j in range(_L):
            srows = jnp.where(lanes == tgt[j], base + i * _L + j, srows)
        dlist_v[pl.ds(i * _L, _L)] = srows
        scats[i] = pltpu.async_copy(
            pat_v.at[pl.ds(_L - z, _L)],
            out_hbm.at[dlist_v.at[pl.ds(i * _L, _L)]],
            ssem0 if i % 2 == 0 else ssem1)
    for i in range(n_ch - _W, n_ch):
        scats[i].wait()


def kernel(inputs, table):
    B, L = inputs.shape
    H = table.shape[1]
    n = B * L
    idx = inputs.reshape(n)
    pattern = jnp.repeat(table, _L, axis=0).reshape(2 * _L, 8, H // 8)
    mesh = plsc.VectorSubcoreMesh(core_axis_name="c", subcore_axis_name="s")
    b_per_w = n // _NW
    k = functools.partial(
        pl.kernel,
        mesh=mesh,
        out_type=jax.ShapeDtypeStruct((n, 8, H // 8), jnp.float32),
        scratch_types=[
            pltpu.VMEM((b_per_w,), jnp.int32),
            pltpu.VMEM((b_per_w + _L,), jnp.int32),
            pltpu.VMEM((2 * _L, 8, H // 8), jnp.float32),
            pltpu.SemaphoreType.DMA,
            pltpu.SemaphoreType.DMA,
            pltpu.SemaphoreType.DMA,
        ],
    )(_sc_embed)
    out = k(idx, pattern)
    return out.reshape(B, L, H)
